# Initial kernel scaffold; baseline (speedup 1.0000x reference)
import jax
import jax.numpy as jnp
from jax.experimental import pallas as pl


def kernel(user_emb, item_emb, edge_rows, edge_cols, edge_vals, u_w, i_w, u_cat_w, i_cat_w):
    raise NotImplementedError("write your pallas kernel here")


# speedup vs baseline: 4.5167x; 4.5167x over previous
import functools

import jax
import jax.numpy as jnp
from jax import lax
from jax.experimental import pallas as pl
from jax.experimental.pallas import tpu as pltpu
from jax.experimental.pallas import tpu_sc as plsc

U = 10000
IT = 10000
D = 128
E = 320000
B = 3
L = 2

NT = 16
K = 128
NCHUNK = 158
EPT = NCHUNK * K
EP = EPT * NT
NP = 10240
RPT = NP // NT
ZR = 64
NJ = D // 16


_mesh = plsc.VectorSubcoreMesh(core_axis_name="c", subcore_axis_name="s")


@functools.partial(
    pl.kernel,
    mesh=_mesh,
    out_type=jax.ShapeDtypeStruct((2, B, NP, D), jnp.float32),
    scratch_types=[
        pltpu.VMEM((3, K), jnp.int32),
        pltpu.VMEM((3, K), jnp.int32),
        pltpu.VMEM((2 * K, D), jnp.float32),
        pltpu.VMEM((ZR, D), jnp.float32),
        pltpu.VMEM_SHARED((NP, D), jnp.float32),
        pltpu.SemaphoreType.DMA,
        pltpu.SemaphoreType.DMA,
        pltpu.SemaphoreType.DMA,
        pltpu.SemaphoreType.DMA,
    ],
)
def _sc_spmm(x_hbm, e6, out5,
             ib0, ib1, gbuf, zbuf, acc, gsem0, gsem1, isem0, isem1):
    c = lax.axis_index("c")
    s = lax.axis_index("s")

    def _zero_zbuf(e, carry):
        for j in range(NJ):
            zbuf[e, pl.ds(j * 16, 16)] = jnp.zeros((16,), jnp.float32)
        return carry

    lax.fori_loop(0, ZR, _zero_zbuf, 0)

    def idx_load(b, i, ib, isem):
        pltpu.async_copy(e6.at[c, b, s, i], ib, isem)

    def idx_wait(b, ib, isem):
        pltpu.make_async_copy(e6.at[c, b, s, 0], ib, isem).wait()

    def gather_start(ib, boff, gsem):
        pltpu.async_copy(x_hbm.at[ib.at[0]], gbuf.at[pl.ds(boff, K)], gsem)

    def gather_wait(ib, boff, gsem):
        pltpu.make_async_copy(
            x_hbm.at[ib.at[0]], gbuf.at[pl.ds(boff, K)], gsem).wait()

    def scale_scatter(ib, boff):
        def body(t, carry):
            vv = ib[2, pl.ds(t * 16, 16)]
            for u in range(16):
                v = lax.bitcast_convert_type(vv[u], jnp.float32)
                r = boff + t * 16 + u
                for j in range(NJ):
                    sl = pl.ds(j * 16, 16)
                    gbuf[r, sl] = gbuf[r, sl] * v
            return carry

        lax.fori_loop(0, K // 16, body, 0)
        pltpu.sync_copy(gbuf.at[pl.ds(boff, K)], acc.at[ib.at[1]], add=True)

    for b in range(B):
        for z in range(RPT // ZR):
            pltpu.sync_copy(zbuf, acc.at[pl.ds(s * RPT + z * ZR, ZR)])
        plsc.subcore_barrier()

        pltpu.sync_copy(e6.at[c, b, s, 0], ib0)
        gather_start(ib0, 0, gsem0)
        idx_load(b, 1, ib1, isem1)

        def pair(k, carry):
            i0 = 2 * k
            gather_wait(ib0, 0, gsem0)
            idx_wait(b, ib1, isem1)
            gather_start(ib1, K, gsem1)
            scale_scatter(ib0, 0)
            idx_load(b, i0 + 2, ib0, isem0)
            gather_wait(ib1, K, gsem1)
            idx_wait(b, ib0, isem0)
            gather_start(ib0, 0, gsem0)
            scale_scatter(ib1, K)
            idx_load(b, i0 + 3, ib1, isem1)
            return carry

        lax.fori_loop(0, NCHUNK // 2 - 1, pair, 0)
        gather_wait(ib0, 0, gsem0)
        idx_wait(b, ib1, isem1)
        gather_start(ib1, K, gsem1)
        scale_scatter(ib0, 0)
        gather_wait(ib1, K, gsem1)
        scale_scatter(ib1, K)

        plsc.subcore_barrier()
        pltpu.sync_copy(acc.at[pl.ds(s * RPT, RPT)],
                        out5.at[c, b, pl.ds(s * RPT, RPT)])


_TCR = 512


def _tc_layer_body(xs_ref, w_ref, new_ref, all_ref):
    w = w_ref[...]
    ps = []
    for b in range(B):
        p = jnp.dot(xs_ref[b], w, preferred_element_type=jnp.float32)
        all_ref[b] = jax.nn.sigmoid(p)
        ps.append(p)
    new_ref[...] = jax.nn.sigmoid((ps[0] + ps[1] + ps[2]) * (1.0 / 3.0))


def _tc_layer(xs, w):
    n = xs.shape[1]
    return pl.pallas_call(
        _tc_layer_body,
        grid=(n // _TCR,),
        in_specs=[
            pl.BlockSpec((B, _TCR, D), lambda i: (0, i, 0)),
            pl.BlockSpec((D, D), lambda i: (0, 0)),
        ],
        out_specs=[
            pl.BlockSpec((_TCR, D), lambda i: (i, 0)),
            pl.BlockSpec((B, _TCR, D), lambda i: (0, i, 0)),
        ],
        out_shape=[
            jax.ShapeDtypeStruct((n, D), jnp.float32),
            jax.ShapeDtypeStruct((B, n, D), jnp.float32),
        ],
    )(xs, w)


def _tc_cat_body(x1_ref, x2_ref, xs1_ref, xs2_ref, w_ref, emb_ref, embs_ref):
    w1 = w_ref[pl.ds(0, D), :]
    w2 = w_ref[pl.ds(D, D), :]
    emb_ref[...] = (jnp.dot(x1_ref[...], w1, preferred_element_type=jnp.float32)
                    + jnp.dot(x2_ref[...], w2, preferred_element_type=jnp.float32))
    for b in range(B):
        embs_ref[b] = (jnp.dot(xs1_ref[b], w1, preferred_element_type=jnp.float32)
                       + jnp.dot(xs2_ref[b], w2, preferred_element_type=jnp.float32))


def _tc_cat(x1, x2, xs1, xs2, cat_w):
    n = x1.shape[0]
    return pl.pallas_call(
        _tc_cat_body,
        grid=(n // _TCR,),
        in_specs=[
            pl.BlockSpec((_TCR, D), lambda i: (i, 0)),
            pl.BlockSpec((_TCR, D), lambda i: (i, 0)),
            pl.BlockSpec((B, _TCR, D), lambda i: (0, i, 0)),
            pl.BlockSpec((B, _TCR, D), lambda i: (0, i, 0)),
            pl.BlockSpec((L * D, D), lambda i: (0, 0)),
        ],
        out_specs=[
            pl.BlockSpec((_TCR, D), lambda i: (i, 0)),
            pl.BlockSpec((B, _TCR, D), lambda i: (0, i, 0)),
        ],
        out_shape=[
            jax.ShapeDtypeStruct((n, D), jnp.float32),
            jax.ShapeDtypeStruct((B, n, D), jnp.float32),
        ],
    )(x1, x2, xs1, xs2, cat_w)


def kernel(user_emb, item_emb, edge_rows, edge_cols, edge_vals,
           u_w, i_w, u_cat_w, i_cat_w):
    pad = EP - E
    r4 = jnp.pad(edge_rows, ((0, 0), (0, pad))).reshape(B, NT, NCHUNK, K)
    c4 = jnp.pad(edge_cols, ((0, 0), (0, pad))).reshape(B, NT, NCHUNK, K)
    v4 = lax.bitcast_convert_type(
        jnp.pad(edge_vals, ((0, 0), (0, pad))), jnp.int32
    ).reshape(B, NT, NCHUNK, K)
    e6 = jnp.stack([
        jnp.stack([c4 + NP, r4, v4], axis=-2),
        jnp.stack([r4, c4, v4], axis=-2),
    ])

    zpad = ((0, NP - U), (0, 0))
    ue = jnp.pad(user_emb, zpad)
    ie = jnp.pad(item_emb, zpad)
    news, alls = [], []
    for l in range(L):
        x_all = jnp.concatenate([ue, ie], axis=0)
        out5 = _sc_spmm(x_all, e6)
        ue_s, ie_s = out5[0], out5[1]
        ue, ue_all = _tc_layer(ue_s, u_w[l])
        ie, ie_all = _tc_layer(ie_s, i_w[l])
        news.append((ue, ie))
        alls.append((ue_all, ie_all))

    u_emb, u_embs = _tc_cat(news[0][0], news[1][0], alls[0][0], alls[1][0], u_cat_w)
    i_emb, i_embs = _tc_cat(news[0][1], news[1][1], alls[0][1], alls[1][1], i_cat_w)
    return (u_emb[:U], i_emb[:IT], u_embs[:, :U], i_embs[:, :IT])
